# Initial kernel scaffold; baseline (speedup 1.0000x reference)
import jax
import jax.numpy as jnp
from jax.experimental import pallas as pl


def kernel(p_x, v_x, p_lin_W, p_lin_b, p_g1_W, p_g1_b, p_g2_W, p_g2_b, v_lin_W, v_lin_b, v_g1_W, v_g1_b, v_g2_W, v_g2_b, att_Wq, att_Wk, p_edge_index, p_batch, v_edge_index, v_batch):
    raise NotImplementedError("write your pallas kernel here")


# speedup vs baseline: 1.0432x; 1.0432x over previous
import functools
from typing import Any

import jax
import jax.numpy as jnp
import numpy as np
from jax.experimental import pallas as pl
from jax.experimental.pallas import tpu as pltpu

B = 8
EMB = 128
H = 4
P_N, P_E = 10000, 320000
V_N, V_E = 2048, 8192
P_MAX, V_MAX = 2048, 384


def _gcn_conv_xla(x, edge_index, W, b):
    N = x.shape[0]
    loop = jnp.arange(N)
    src = jnp.concatenate([edge_index[0], loop])
    dst = jnp.concatenate([edge_index[1], loop])
    deg = jnp.bincount(dst, length=N).astype(x.dtype)
    dinv = jax.lax.rsqrt(jnp.maximum(deg, 1.0))
    xw = x @ W
    msg = xw[src] * (dinv[src] * dinv[dst])[:, None]
    return jax.ops.segment_sum(msg, dst, num_segments=N) + b


def _encode_xla(x, edge_index, lin_W, lin_b, g1_W, g1_b, g2_W, g2_b):
    h0 = x @ lin_W + lin_b
    h = jax.nn.relu(_gcn_conv_xla(h0, edge_index, g1_W, g1_b))
    h = _gcn_conv_xla(h, edge_index, g2_W, g2_b)
    return h0, h


def _finalize_body(h2p_ref, h0p_ref, h2v_ref, h0v_ref, pb_ref, vb_ref,
                   wq_ref, wk_ref,
                   fusion_ref, compat_ref, mask_ref):
    b = pl.program_id(0)
    bf = jnp.float32(b)

    pb = pb_ref[...]
    vb = vb_ref[...]
    pm = (pb == bf).astype(jnp.float32)
    vm = (vb == bf).astype(jnp.float32)
    p_cnt = jnp.sum(pm)
    v_cnt = jnp.sum(vm)
    p_start = jnp.sum((pb >= 0.0) & (pb < bf)).astype(jnp.int32)
    v_start = jnp.sum((vb >= 0.0) & (vb < bf)).astype(jnp.int32)

    h2p = h2p_ref[...]
    h2v = h2v_ref[...]
    p_gsum = jax.lax.dot_general(pm, h2p, (((1,), (0,)), ((), ())),
                                 preferred_element_type=jnp.float32)
    v_gsum = jax.lax.dot_general(vm, h2v, (((1,), (0,)), ((), ())),
                                 preferred_element_type=jnp.float32)
    p_g = p_gsum / jnp.maximum(p_cnt, 1.0)
    v_g = v_gsum / jnp.maximum(v_cnt, 1.0)

    fusion_ref[...] = ((p_g + v_g) / 2.0)[None]

    ps = p_start
    vs = v_start
    p_rows = h2p_ref[pl.ds(ps, P_MAX), :] + h0p_ref[pl.ds(ps, P_MAX), :]
    v_rows = h2v_ref[pl.ds(vs, V_MAX), :] + h0v_ref[pl.ds(vs, V_MAX), :]
    p_cnt_i = jnp.minimum(p_cnt, float(P_MAX)).astype(jnp.int32)
    v_cnt_i = jnp.minimum(v_cnt, float(V_MAX)).astype(jnp.int32)
    p_iota = jax.lax.broadcasted_iota(jnp.int32, (P_MAX, 1), 0)
    v_iota = jax.lax.broadcasted_iota(jnp.int32, (V_MAX, 1), 0)
    p_valid = (p_iota < p_cnt_i).astype(jnp.float32)
    v_valid = (v_iota < v_cnt_i).astype(jnp.float32)
    P_blk = p_rows * p_valid + p_g
    V_blk = v_rows * v_valid + v_g

    dh = EMB // H
    scale = 1.0 / (H * np.sqrt(dh))
    A = jax.lax.dot_general(wq_ref[...], wk_ref[...], (((1,), (1,)), ((), ())),
                            preferred_element_type=jnp.float32) * scale
    PA = jax.lax.dot_general(P_blk, A, (((1,), (0,)), ((), ())),
                             preferred_element_type=jnp.float32)
    compat = jax.lax.dot_general(PA, V_blk, (((1,), (1,)), ((), ())),
                                 preferred_element_type=jnp.float32)
    compat_ref[...] = compat[None]

    v_iota_row = jax.lax.broadcasted_iota(jnp.int32, (P_MAX, V_MAX), 1)
    mask_ref[...] = (v_iota_row < v_cnt_i)[None]


def _finalize(h2p, h0p, h2v, h0v, p_batch_f, v_batch_f, att_Wq, att_Wk,
              interpret=False):
    Np = P_N + P_MAX
    Nv = V_N + V_MAX
    full = lambda shape: pl.BlockSpec(shape, lambda b: (0,) * len(shape))
    return pl.pallas_call(
        _finalize_body,
        grid=(B,),
        in_specs=[
            full((Np, EMB)), full((Np, EMB)),
            full((Nv, EMB)), full((Nv, EMB)),
            full((1, Np)), full((1, Nv)),
            full((EMB, EMB)), full((EMB, EMB)),
        ],
        out_specs=[
            pl.BlockSpec((1, 1, EMB), lambda b: (b, 0, 0)),
            pl.BlockSpec((1, P_MAX, V_MAX), lambda b: (b, 0, 0)),
            pl.BlockSpec((1, P_MAX, V_MAX), lambda b: (b, 0, 0)),
        ],
        out_shape=[
            jax.ShapeDtypeStruct((B, 1, EMB), jnp.float32),
            jax.ShapeDtypeStruct((B, P_MAX, V_MAX), jnp.float32),
            jax.ShapeDtypeStruct((B, P_MAX, V_MAX), jnp.bool_),
        ],
        interpret=interpret,
    )(h2p, h0p, h2v, h0v, p_batch_f, v_batch_f, att_Wq, att_Wk)


def kernel(p_x, v_x, p_lin_W, p_lin_b, p_g1_W, p_g1_b, p_g2_W, p_g2_b,
           v_lin_W, v_lin_b, v_g1_W, v_g1_b, v_g2_W, v_g2_b,
           att_Wq, att_Wk, p_edge_index, p_batch, v_edge_index, v_batch):
    h0p, h2p = _encode_xla(p_x, p_edge_index, p_lin_W, p_lin_b,
                           p_g1_W, p_g1_b, p_g2_W, p_g2_b)
    h0v, h2v = _encode_xla(v_x, v_edge_index, v_lin_W, v_lin_b,
                           v_g1_W, v_g1_b, v_g2_W, v_g2_b)

    pad_rows = lambda a, n: jnp.pad(a, ((0, n), (0, 0)))
    h2p_p = pad_rows(h2p, P_MAX)
    h0p_p = pad_rows(h0p, P_MAX)
    h2v_p = pad_rows(h2v, V_MAX)
    h0v_p = pad_rows(h0v, V_MAX)
    pbf = jnp.pad(p_batch.astype(jnp.float32), (0, P_MAX),
                  constant_values=-1.0)[None]
    vbf = jnp.pad(v_batch.astype(jnp.float32), (0, V_MAX),
                  constant_values=-1.0)[None]

    fusion, compat, att_mask = _finalize(h2p_p, h0p_p, h2v_p, h0v_p,
                                         pbf, vbf, att_Wq, att_Wk)
    return fusion[:, 0, :], compat, att_mask


# speedup vs baseline: 10.5665x; 10.1285x over previous
import functools

import jax
import jax.numpy as jnp
import numpy as np
from jax import lax
from jax.experimental import pallas as pl
from jax.experimental.pallas import tpu as pltpu
from jax.experimental.pallas import tpu_sc as plsc

B = 8
EMB = 128
H = 4
P_N, P_E = 10000, 320000
V_N, V_E = 2048, 8192
P_MAX, V_MAX = 2048, 384

PAD_NODE = P_N
V_BASE = 10240
NODES = V_BASE + V_N
FIN_ROWS = V_BASE + V_N + V_MAX

NC, NS, L = 2, 16, 16
NW = NC * NS
CH = 128
E_ALL = P_E + V_E
T = 2592
E_PAD = T * CH
NT = T // NW
ROWS_PER_TILE = NODES // NS


def _sc_mesh():
    return plsc.VectorSubcoreMesh(core_axis_name="c", subcore_axis_name="s",
                                  num_cores=NC, num_subcores=NS)


def _zero_fill_128w(zbuf):
    z16 = jnp.zeros((L,), jnp.float32)

    def body(i, _):
        for g in range(8):
            zbuf[i, pl.ds(g * L, L)] = z16
        return 0
    lax.fori_loop(0, 128, body, 0)


def _deg_body(dstT, deg_out, bins, buf, didx):
    c = lax.axis_index("c")
    s = lax.axis_index("s")
    wid = s * NC + c

    _zero_fill_128w(buf)
    r0 = s * ROWS_PER_TILE
    for j in range(ROWS_PER_TILE // 128):
        pltpu.sync_copy(buf, bins.at[pl.ds(r0 + j * 128, 128)])

    one16 = jnp.full((L,), 1.0, jnp.float32)

    def fill(i, _):
        for g in range(EMB // L):
            buf[i, pl.ds(g * L, L)] = one16
        return 0
    lax.fori_loop(0, 128, fill, 0)
    plsc.subcore_barrier()

    pltpu.sync_copy(dstT.at[wid], didx)

    def scat(j, _):
        pltpu.sync_copy(buf, bins.at[didx.at[j]], add=True)
        return 0
    lax.fori_loop(0, NT, scat, 0)
    plsc.subcore_barrier()

    pltpu.sync_copy(bins.at[pl.ds(r0, ROWS_PER_TILE)],
                    deg_out.at[c, pl.ds(r0, ROWS_PER_TILE)])


def _sc_degrees(dstT):
    kern = pl.kernel(
        _deg_body,
        out_type=jax.ShapeDtypeStruct((NC, NODES, EMB), jnp.float32),
        mesh=_sc_mesh(),
        scratch_types=[
            pltpu.VMEM_SHARED((NODES, EMB), jnp.float32),
            pltpu.VMEM((128, EMB), jnp.float32),
            pltpu.VMEM((NT, CH), jnp.int32),
        ],
    )
    return kern(dstT)


def _conv_body(y, src1d, dstT, S_out, acc, rows, sidx, didx, sem):
    c = lax.axis_index("c")
    s = lax.axis_index("s")
    wid = s * NC + c

    _zero_fill_128w(rows)
    r0 = s * ROWS_PER_TILE
    for j in range(ROWS_PER_TILE // 128):
        pltpu.sync_copy(rows, acc.at[pl.ds(r0 + j * 128, 128)])
    plsc.subcore_barrier()

    pltpu.sync_copy(dstT.at[wid], didx)
    base = wid * NT * CH

    def step(j, _):
        pltpu.sync_copy(src1d.at[pl.ds(base + j * CH, CH)], sidx)
        pltpu.async_copy(y.at[sidx], rows, sem).wait()
        pltpu.sync_copy(rows, acc.at[didx.at[j]], add=True)
        return 0
    lax.fori_loop(0, NT, step, 0)
    plsc.subcore_barrier()

    pltpu.sync_copy(acc.at[pl.ds(r0, ROWS_PER_TILE)],
                    S_out.at[c, pl.ds(r0, ROWS_PER_TILE)])


def _sc_edge_aggregate(y, src1d, dstT):
    kern = pl.kernel(
        _conv_body,
        out_type=jax.ShapeDtypeStruct((NC, NODES, EMB), jnp.float32),
        mesh=_sc_mesh(),
        scratch_types=[
            pltpu.VMEM_SHARED((NODES, EMB), jnp.float32),
            pltpu.VMEM((CH, EMB), jnp.float32),
            pltpu.VMEM((CH,), jnp.int32),
            pltpu.VMEM((NT, CH), jnp.int32),
            pltpu.SemaphoreType.DMA,
        ],
    )
    return kern(y, src1d, dstT)


NODE_BLK = 1024
N_BLKS = NODES // NODE_BLK
P_BLKS = V_BASE // NODE_BLK


def _wsel(k):
    return jnp.where(k < P_BLKS, 0, 1)


def _tc_a_body(x_ref, linW_ref, linb_ref, g1W_ref, deg_ref,
               h0_ref, y1_ref, dinv_ref):
    x = x_ref[...]
    h0 = jax.lax.dot_general(x, linW_ref[0], (((1,), (0,)), ((), ())),
                             preferred_element_type=jnp.float32) + linb_ref[0]
    deg = (jnp.sum(deg_ref[0] + deg_ref[1], axis=-1, keepdims=True)
           * (1.0 / EMB) + 1.0)
    dinv = lax.rsqrt(jnp.maximum(deg, 1.0))
    xw = jax.lax.dot_general(h0, g1W_ref[0], (((1,), (0,)), ((), ())),
                             preferred_element_type=jnp.float32)
    h0_ref[...] = h0
    y1_ref[...] = xw * dinv
    dinv_ref[...] = dinv


def _tc_a(x_all, linW, linb, g1W, deg):
    return pl.pallas_call(
        _tc_a_body,
        grid=(N_BLKS,),
        in_specs=[
            pl.BlockSpec((NODE_BLK, EMB), lambda k: (k, 0)),
            pl.BlockSpec((1, EMB, EMB), lambda k: (_wsel(k), 0, 0)),
            pl.BlockSpec((1, 1, EMB), lambda k: (_wsel(k), 0, 0)),
            pl.BlockSpec((1, EMB, EMB), lambda k: (_wsel(k), 0, 0)),
            pl.BlockSpec((2, NODE_BLK, EMB), lambda k: (0, k, 0)),
        ],
        out_specs=[
            pl.BlockSpec((NODE_BLK, EMB), lambda k: (k, 0)),
            pl.BlockSpec((NODE_BLK, EMB), lambda k: (k, 0)),
            pl.BlockSpec((NODE_BLK, 1), lambda k: (k, 0)),
        ],
        out_shape=[
            jax.ShapeDtypeStruct((NODES, EMB), jnp.float32),
            jax.ShapeDtypeStruct((NODES, EMB), jnp.float32),
            jax.ShapeDtypeStruct((NODES, 1), jnp.float32),
        ],
    )(x_all, linW, linb, g1W, deg)


def _tc_b_body(S_ref, y1_ref, dinv_ref, g2W_ref, b1_ref, y2_ref):
    dinv = dinv_ref[...]
    h1 = jax.nn.relu(dinv * (S_ref[0] + S_ref[1] + y1_ref[...]) + b1_ref[0])
    xw = jax.lax.dot_general(h1, g2W_ref[0], (((1,), (0,)), ((), ())),
                             preferred_element_type=jnp.float32)
    y2_ref[...] = xw * dinv


def _tc_b(S1, y1, dinv, g2W, b1):
    return pl.pallas_call(
        _tc_b_body,
        grid=(N_BLKS,),
        in_specs=[
            pl.BlockSpec((2, NODE_BLK, EMB), lambda k: (0, k, 0)),
            pl.BlockSpec((NODE_BLK, EMB), lambda k: (k, 0)),
            pl.BlockSpec((NODE_BLK, 1), lambda k: (k, 0)),
            pl.BlockSpec((1, EMB, EMB), lambda k: (_wsel(k), 0, 0)),
            pl.BlockSpec((1, 1, EMB), lambda k: (_wsel(k), 0, 0)),
        ],
        out_specs=pl.BlockSpec((NODE_BLK, EMB), lambda k: (k, 0)),
        out_shape=jax.ShapeDtypeStruct((NODES, EMB), jnp.float32),
    )(S1, y1, dinv, g2W, b1)


def _tc_c_body(S_ref, y2_ref, dinv_ref, b2_ref, h2_ref):
    h2_ref[...] = (dinv_ref[...] * (S_ref[0] + S_ref[1] + y2_ref[...])
                   + b2_ref[0])


def _tc_c(S2, y2, dinv, b2):
    return pl.pallas_call(
        _tc_c_body,
        grid=(N_BLKS,),
        in_specs=[
            pl.BlockSpec((2, NODE_BLK, EMB), lambda k: (0, k, 0)),
            pl.BlockSpec((NODE_BLK, EMB), lambda k: (k, 0)),
            pl.BlockSpec((NODE_BLK, 1), lambda k: (k, 0)),
            pl.BlockSpec((1, 1, EMB), lambda k: (_wsel(k), 0, 0)),
        ],
        out_specs=pl.BlockSpec((NODE_BLK, EMB), lambda k: (k, 0)),
        out_shape=jax.ShapeDtypeStruct((NODES, EMB), jnp.float32),
    )(S2, y2, dinv, b2)


def _finalize_body(h2_ref, h0_ref, pb_ref, vb_ref, wq_ref, wk_ref,
                   fusion_ref, compat_ref, mask_ref):
    b = pl.program_id(0)
    bf = jnp.float32(b)

    pb = pb_ref[...]
    vb = vb_ref[...]
    pm = (pb == bf).astype(jnp.float32)
    vm = (vb == bf).astype(jnp.float32)
    p_cnt = jnp.sum(pm)
    v_cnt = jnp.sum(vm)
    p_start = jnp.sum((pb >= 0.0) & (pb < bf)).astype(jnp.int32)
    v_start = jnp.sum((vb >= 0.0) & (vb < bf)).astype(jnp.int32) + V_BASE

    h2 = h2_ref[...]
    p_g = jax.lax.dot_general(pm, h2, (((1,), (0,)), ((), ())),
                              preferred_element_type=jnp.float32
                              ) / jnp.maximum(p_cnt, 1.0)
    v_g = jax.lax.dot_general(vm, h2, (((1,), (0,)), ((), ())),
                              preferred_element_type=jnp.float32
                              ) / jnp.maximum(v_cnt, 1.0)
    fusion_ref[...] = ((p_g + v_g) / 2.0)[None]

    p_rows = h2_ref[pl.ds(p_start, P_MAX), :] + h0_ref[pl.ds(p_start, P_MAX), :]
    v_rows = h2_ref[pl.ds(v_start, V_MAX), :] + h0_ref[pl.ds(v_start, V_MAX), :]
    p_cnt_i = jnp.minimum(p_cnt, float(P_MAX)).astype(jnp.int32)
    v_cnt_i = jnp.minimum(v_cnt, float(V_MAX)).astype(jnp.int32)
    p_valid = (jax.lax.broadcasted_iota(jnp.int32, (P_MAX, 1), 0)
               < p_cnt_i).astype(jnp.float32)
    v_valid = (jax.lax.broadcasted_iota(jnp.int32, (V_MAX, 1), 0)
               < v_cnt_i).astype(jnp.float32)
    P_blk = p_rows * p_valid + p_g
    V_blk = v_rows * v_valid + v_g

    dh = EMB // H
    scale = 1.0 / (H * np.sqrt(dh))
    A = jax.lax.dot_general(wq_ref[...], wk_ref[...], (((1,), (1,)), ((), ())),
                            preferred_element_type=jnp.float32) * scale
    PA = jax.lax.dot_general(P_blk, A, (((1,), (0,)), ((), ())),
                             preferred_element_type=jnp.float32)
    compat = jax.lax.dot_general(PA, V_blk, (((1,), (1,)), ((), ())),
                                 preferred_element_type=jnp.float32)
    compat_ref[...] = compat[None]

    v_iota_row = jax.lax.broadcasted_iota(jnp.int32, (P_MAX, V_MAX), 1)
    mask_ref[...] = (v_iota_row < v_cnt_i)[None]


def _finalize(h2f, h0f, pbf, vbf, att_Wq, att_Wk):
    full = lambda shape: pl.BlockSpec(shape, lambda b: (0,) * len(shape))
    return pl.pallas_call(
        _finalize_body,
        grid=(B,),
        in_specs=[
            full((FIN_ROWS, EMB)), full((FIN_ROWS, EMB)),
            full((1, FIN_ROWS)), full((1, FIN_ROWS)),
            full((EMB, EMB)), full((EMB, EMB)),
        ],
        out_specs=[
            pl.BlockSpec((1, 1, EMB), lambda b: (b, 0, 0)),
            pl.BlockSpec((1, P_MAX, V_MAX), lambda b: (b, 0, 0)),
            pl.BlockSpec((1, P_MAX, V_MAX), lambda b: (b, 0, 0)),
        ],
        out_shape=[
            jax.ShapeDtypeStruct((B, 1, EMB), jnp.float32),
            jax.ShapeDtypeStruct((B, P_MAX, V_MAX), jnp.float32),
            jax.ShapeDtypeStruct((B, P_MAX, V_MAX), jnp.bool_),
        ],
    )(h2f, h0f, pbf, vbf, att_Wq, att_Wk)


def kernel(p_x, v_x, p_lin_W, p_lin_b, p_g1_W, p_g1_b, p_g2_W, p_g2_b,
           v_lin_W, v_lin_b, v_g1_W, v_g1_b, v_g2_W, v_g2_b,
           att_Wq, att_Wk, p_edge_index, p_batch, v_edge_index, v_batch):
    f32 = jnp.float32
    i32 = jnp.int32

    x_all = jnp.concatenate([
        jnp.pad(p_x, ((0, V_BASE - P_N), (0, 0))), v_x], axis=0)
    pad_idx = jnp.full((E_PAD - E_ALL,), PAD_NODE, i32)
    src_all = jnp.concatenate([
        p_edge_index[0].astype(i32), v_edge_index[0].astype(i32) + V_BASE,
        pad_idx])
    dst_all = jnp.concatenate([
        p_edge_index[1].astype(i32), v_edge_index[1].astype(i32) + V_BASE,
        pad_idx]).reshape(NW, NT, CH)

    linW = jnp.stack([p_lin_W, v_lin_W])
    linb = jnp.stack([p_lin_b, v_lin_b])[:, None, :]
    g1W = jnp.stack([p_g1_W, v_g1_W])
    b1 = jnp.stack([p_g1_b, v_g1_b])[:, None, :]
    g2W = jnp.stack([p_g2_W, v_g2_W])
    b2 = jnp.stack([p_g2_b, v_g2_b])[:, None, :]

    deg_bins = _sc_degrees(dst_all)

    h0, y1, dinv = _tc_a(x_all, linW, linb, g1W, deg_bins)
    S1 = _sc_edge_aggregate(y1, src_all, dst_all)
    y2 = _tc_b(S1, y1, dinv, g2W, b1)
    S2 = _sc_edge_aggregate(y2, src_all, dst_all)
    h2 = _tc_c(S2, y2, dinv, b2)

    h2f = jnp.pad(h2, ((0, FIN_ROWS - NODES), (0, 0)))
    h0f = jnp.pad(h0, ((0, FIN_ROWS - NODES), (0, 0)))
    neg = jnp.full((V_BASE - P_N,), -1.0, f32)
    pbf = jnp.concatenate([
        p_batch.astype(f32), neg,
        jnp.full((FIN_ROWS - V_BASE,), -1.0, f32)])[None]
    vbf = jnp.concatenate([
        jnp.full((V_BASE,), -1.0, f32), v_batch.astype(f32),
        jnp.full((FIN_ROWS - NODES,), -1.0, f32)])[None]

    fusion, compat, att_mask = _finalize(h2f, h0f, pbf, vbf, att_Wq, att_Wk)
    return fusion[:, 0, :], compat, att_mask


# speedup vs baseline: 15.4809x; 1.4651x over previous
import functools

import jax
import jax.numpy as jnp
import numpy as np
from jax import lax
from jax.experimental import pallas as pl
from jax.experimental.pallas import tpu as pltpu
from jax.experimental.pallas import tpu_sc as plsc

B = 8
EMB = 128
H = 4
P_N, P_E = 10000, 320000
V_N, V_E = 2048, 8192
P_MAX, V_MAX = 2048, 384

PAD_NODE = P_N
V_BASE = 10240
NODES = V_BASE + V_N
FIN_ROWS = V_BASE + V_N + V_MAX

NC, NS, L = 2, 16, 16
NW = NC * NS
E_ALL = P_E + V_E
ROWS_PER_TILE = NODES // NS

CH = 128
T = 2592
E_PAD = T * CH
NT = T // NW

CCH = 112
CNT = 92
CE_PAD = NW * CNT * CCH


def _sc_mesh():
    return plsc.VectorSubcoreMesh(core_axis_name="c", subcore_axis_name="s",
                                  num_cores=NC, num_subcores=NS)


def _zero_fill_128w(zbuf):
    z16 = jnp.zeros((L,), jnp.float32)

    def body(i, _):
        for g in range(8):
            zbuf[i, pl.ds(g * L, L)] = z16
        return 0
    lax.fori_loop(0, 128, body, 0)


def _deg_body(dstT, deg_out, bins, buf, didx):
    c = lax.axis_index("c")
    s = lax.axis_index("s")
    wid = s * NC + c

    _zero_fill_128w(buf)
    r0 = s * ROWS_PER_TILE
    for j in range(ROWS_PER_TILE // 128):
        pltpu.sync_copy(buf, bins.at[pl.ds(r0 + j * 128, 128)])

    one16 = jnp.full((L,), 1.0, jnp.float32)

    def fill(i, _):
        for g in range(EMB // L):
            buf[i, pl.ds(g * L, L)] = one16
        return 0
    lax.fori_loop(0, 128, fill, 0)
    plsc.subcore_barrier()

    pltpu.sync_copy(dstT.at[wid], didx)

    def scat(j, _):
        pltpu.sync_copy(buf, bins.at[didx.at[j]], add=True)
        return 0
    lax.fori_loop(0, NT, scat, 0)
    plsc.subcore_barrier()

    pltpu.sync_copy(bins.at[pl.ds(r0, ROWS_PER_TILE)],
                    deg_out.at[c, pl.ds(r0, ROWS_PER_TILE)])


def _sc_degrees(dstT):
    kern = pl.kernel(
        _deg_body,
        out_type=jax.ShapeDtypeStruct((NC, NODES, EMB), jnp.float32),
        mesh=_sc_mesh(),
        scratch_types=[
            pltpu.VMEM_SHARED((NODES, EMB), jnp.float32),
            pltpu.VMEM((128, EMB), jnp.float32),
            pltpu.VMEM((NT, CH), jnp.int32),
        ],
    )
    return kern(dstT)


def _conv_body(y, src1d, dst1d, S_out, acc,
               rows0, rows1, sidx0, sidx1, didx0, didx1, sem0, sem1):
    c = lax.axis_index("c")
    s = lax.axis_index("s")
    wid = s * NC + c

    def zfill(i, _):
        for g in range(EMB // L):
            rows0[i, pl.ds(g * L, L)] = jnp.zeros((L,), jnp.float32)
        return 0
    lax.fori_loop(0, CCH, zfill, 0)
    r0 = s * ROWS_PER_TILE
    for off in range(0, ROWS_PER_TILE, CCH):
        size = min(CCH, ROWS_PER_TILE - off)
        pltpu.sync_copy(rows0.at[pl.ds(0, size)],
                        acc.at[pl.ds(r0 + off, size)])
    plsc.subcore_barrier()

    base = wid * CNT * CCH

    def fetch(j, sidx, didx, rows, sem):
        off = base + j * CCH
        pltpu.sync_copy(src1d.at[pl.ds(off, CCH)], sidx)
        pltpu.sync_copy(dst1d.at[pl.ds(off, CCH)], didx)
        pltpu.async_copy(y.at[sidx], rows, sem)

    fetch(0, sidx0, didx0, rows0, sem0)

    def work(i, sidx_c, didx_c, rows_c, sem_c, sidx_n, didx_n, rows_n, sem_n):
        @pl.when(i + 1 < CNT)
        def _():
            fetch(i + 1, sidx_n, didx_n, rows_n, sem_n)
        pltpu.make_async_copy(y.at[sidx_c], rows_c, sem_c).wait()
        pltpu.sync_copy(rows_c, acc.at[didx_c], add=True)

    def step(i, _):
        @pl.when(i % 2 == 0)
        def _():
            work(i, sidx0, didx0, rows0, sem0, sidx1, didx1, rows1, sem1)
        @pl.when(i % 2 == 1)
        def _():
            work(i, sidx1, didx1, rows1, sem1, sidx0, didx0, rows0, sem0)
        return 0
    lax.fori_loop(0, CNT, step, 0)
    plsc.subcore_barrier()

    pltpu.sync_copy(acc.at[pl.ds(r0, ROWS_PER_TILE)],
                    S_out.at[c, pl.ds(r0, ROWS_PER_TILE)])


def _sc_edge_aggregate(y, src1d, dst1d):
    kern = pl.kernel(
        _conv_body,
        out_type=jax.ShapeDtypeStruct((NC, NODES, EMB), jnp.float32),
        mesh=_sc_mesh(),
        scratch_types=[
            pltpu.VMEM_SHARED((NODES, EMB), jnp.float32),
            pltpu.VMEM((CCH, EMB), jnp.float32),
            pltpu.VMEM((CCH, EMB), jnp.float32),
            pltpu.VMEM((CCH,), jnp.int32),
            pltpu.VMEM((CCH,), jnp.int32),
            pltpu.VMEM((CCH,), jnp.int32),
            pltpu.VMEM((CCH,), jnp.int32),
            pltpu.SemaphoreType.DMA,
            pltpu.SemaphoreType.DMA,
        ],
    )
    return kern(y, src1d, dst1d)


NODE_BLK = 1024
N_BLKS = NODES // NODE_BLK
P_BLKS = V_BASE // NODE_BLK


def _wsel(k):
    return jnp.where(k < P_BLKS, 0, 1)


def _tc_a_body(x_ref, linW_ref, linb_ref, g1W_ref, deg_ref,
               h0_ref, y1_ref, dinv_ref):
    x = x_ref[...]
    h0 = jax.lax.dot_general(x, linW_ref[0], (((1,), (0,)), ((), ())),
                             preferred_element_type=jnp.float32) + linb_ref[0]
    deg = (jnp.sum(deg_ref[0] + deg_ref[1], axis=-1, keepdims=True)
           * (1.0 / EMB) + 1.0)
    dinv = lax.rsqrt(jnp.maximum(deg, 1.0))
    xw = jax.lax.dot_general(h0, g1W_ref[0], (((1,), (0,)), ((), ())),
                             preferred_element_type=jnp.float32)
    h0_ref[...] = h0
    y1_ref[...] = xw * dinv
    dinv_ref[...] = dinv


def _tc_a(x_all, linW, linb, g1W, deg):
    return pl.pallas_call(
        _tc_a_body,
        grid=(N_BLKS,),
        in_specs=[
            pl.BlockSpec((NODE_BLK, EMB), lambda k: (k, 0)),
            pl.BlockSpec((1, EMB, EMB), lambda k: (_wsel(k), 0, 0)),
            pl.BlockSpec((1, 1, EMB), lambda k: (_wsel(k), 0, 0)),
            pl.BlockSpec((1, EMB, EMB), lambda k: (_wsel(k), 0, 0)),
            pl.BlockSpec((2, NODE_BLK, EMB), lambda k: (0, k, 0)),
        ],
        out_specs=[
            pl.BlockSpec((NODE_BLK, EMB), lambda k: (k, 0)),
            pl.BlockSpec((NODE_BLK, EMB), lambda k: (k, 0)),
            pl.BlockSpec((NODE_BLK, 1), lambda k: (k, 0)),
        ],
        out_shape=[
            jax.ShapeDtypeStruct((NODES, EMB), jnp.float32),
            jax.ShapeDtypeStruct((NODES, EMB), jnp.float32),
            jax.ShapeDtypeStruct((NODES, 1), jnp.float32),
        ],
    )(x_all, linW, linb, g1W, deg)


def _tc_b_body(S_ref, y1_ref, dinv_ref, g2W_ref, b1_ref, y2_ref):
    dinv = dinv_ref[...]
    h1 = jax.nn.relu(dinv * (S_ref[0] + S_ref[1] + y1_ref[...]) + b1_ref[0])
    xw = jax.lax.dot_general(h1, g2W_ref[0], (((1,), (0,)), ((), ())),
                             preferred_element_type=jnp.float32)
    y2_ref[...] = xw * dinv


def _tc_b(S1, y1, dinv, g2W, b1):
    return pl.pallas_call(
        _tc_b_body,
        grid=(N_BLKS,),
        in_specs=[
            pl.BlockSpec((2, NODE_BLK, EMB), lambda k: (0, k, 0)),
            pl.BlockSpec((NODE_BLK, EMB), lambda k: (k, 0)),
            pl.BlockSpec((NODE_BLK, 1), lambda k: (k, 0)),
            pl.BlockSpec((1, EMB, EMB), lambda k: (_wsel(k), 0, 0)),
            pl.BlockSpec((1, 1, EMB), lambda k: (_wsel(k), 0, 0)),
        ],
        out_specs=pl.BlockSpec((NODE_BLK, EMB), lambda k: (k, 0)),
        out_shape=jax.ShapeDtypeStruct((NODES, EMB), jnp.float32),
    )(S1, y1, dinv, g2W, b1)


def _tc_c_body(S_ref, y2_ref, dinv_ref, b2_ref, h2_ref):
    h2_ref[...] = (dinv_ref[...] * (S_ref[0] + S_ref[1] + y2_ref[...])
                   + b2_ref[0])


def _tc_c(S2, y2, dinv, b2):
    return pl.pallas_call(
        _tc_c_body,
        grid=(N_BLKS,),
        in_specs=[
            pl.BlockSpec((2, NODE_BLK, EMB), lambda k: (0, k, 0)),
            pl.BlockSpec((NODE_BLK, EMB), lambda k: (k, 0)),
            pl.BlockSpec((NODE_BLK, 1), lambda k: (k, 0)),
            pl.BlockSpec((1, 1, EMB), lambda k: (_wsel(k), 0, 0)),
        ],
        out_specs=pl.BlockSpec((NODE_BLK, EMB), lambda k: (k, 0)),
        out_shape=jax.ShapeDtypeStruct((NODES, EMB), jnp.float32),
    )(S2, y2, dinv, b2)


def _finalize_body(h2_ref, h0_ref, pb_ref, vb_ref, wq_ref, wk_ref,
                   fusion_ref, compat_ref, mask_ref):
    b = pl.program_id(0)
    bf = jnp.float32(b)

    pb = pb_ref[...]
    vb = vb_ref[...]
    pm = (pb == bf).astype(jnp.float32)
    vm = (vb == bf).astype(jnp.float32)
    p_cnt = jnp.sum(pm)
    v_cnt = jnp.sum(vm)
    p_start = jnp.sum((pb >= 0.0) & (pb < bf)).astype(jnp.int32)
    v_start = jnp.sum((vb >= 0.0) & (vb < bf)).astype(jnp.int32) + V_BASE

    h2 = h2_ref[...]
    p_g = jax.lax.dot_general(pm, h2, (((1,), (0,)), ((), ())),
                              preferred_element_type=jnp.float32
                              ) / jnp.maximum(p_cnt, 1.0)
    v_g = jax.lax.dot_general(vm, h2, (((1,), (0,)), ((), ())),
                              preferred_element_type=jnp.float32
                              ) / jnp.maximum(v_cnt, 1.0)
    fusion_ref[...] = ((p_g + v_g) / 2.0)[None]

    p_rows = h2_ref[pl.ds(p_start, P_MAX), :] + h0_ref[pl.ds(p_start, P_MAX), :]
    v_rows = h2_ref[pl.ds(v_start, V_MAX), :] + h0_ref[pl.ds(v_start, V_MAX), :]
    p_cnt_i = jnp.minimum(p_cnt, float(P_MAX)).astype(jnp.int32)
    v_cnt_i = jnp.minimum(v_cnt, float(V_MAX)).astype(jnp.int32)
    p_valid = (jax.lax.broadcasted_iota(jnp.int32, (P_MAX, 1), 0)
               < p_cnt_i).astype(jnp.float32)
    v_valid = (jax.lax.broadcasted_iota(jnp.int32, (V_MAX, 1), 0)
               < v_cnt_i).astype(jnp.float32)
    P_blk = p_rows * p_valid + p_g
    V_blk = v_rows * v_valid + v_g

    dh = EMB // H
    scale = 1.0 / (H * np.sqrt(dh))
    A = jax.lax.dot_general(wq_ref[...], wk_ref[...], (((1,), (1,)), ((), ())),
                            preferred_element_type=jnp.float32) * scale
    PA = jax.lax.dot_general(P_blk, A, (((1,), (0,)), ((), ())),
                             preferred_element_type=jnp.float32)
    compat = jax.lax.dot_general(PA, V_blk, (((1,), (1,)), ((), ())),
                                 preferred_element_type=jnp.float32)
    compat_ref[...] = compat[None]

    v_iota_row = jax.lax.broadcasted_iota(jnp.int32, (P_MAX, V_MAX), 1)
    mask_ref[...] = (v_iota_row < v_cnt_i)[None]


def _finalize(h2f, h0f, pbf, vbf, att_Wq, att_Wk):
    full = lambda shape: pl.BlockSpec(shape, lambda b: (0,) * len(shape))
    return pl.pallas_call(
        _finalize_body,
        grid=(B,),
        in_specs=[
            full((FIN_ROWS, EMB)), full((FIN_ROWS, EMB)),
            full((1, FIN_ROWS)), full((1, FIN_ROWS)),
            full((EMB, EMB)), full((EMB, EMB)),
        ],
        out_specs=[
            pl.BlockSpec((1, 1, EMB), lambda b: (b, 0, 0)),
            pl.BlockSpec((1, P_MAX, V_MAX), lambda b: (b, 0, 0)),
            pl.BlockSpec((1, P_MAX, V_MAX), lambda b: (b, 0, 0)),
        ],
        out_shape=[
            jax.ShapeDtypeStruct((B, 1, EMB), jnp.float32),
            jax.ShapeDtypeStruct((B, P_MAX, V_MAX), jnp.float32),
            jax.ShapeDtypeStruct((B, P_MAX, V_MAX), jnp.bool_),
        ],
    )(h2f, h0f, pbf, vbf, att_Wq, att_Wk)


def kernel(p_x, v_x, p_lin_W, p_lin_b, p_g1_W, p_g1_b, p_g2_W, p_g2_b,
           v_lin_W, v_lin_b, v_g1_W, v_g1_b, v_g2_W, v_g2_b,
           att_Wq, att_Wk, p_edge_index, p_batch, v_edge_index, v_batch):
    f32 = jnp.float32
    i32 = jnp.int32

    x_all = jnp.concatenate([
        jnp.pad(p_x, ((0, V_BASE - P_N), (0, 0))), v_x], axis=0)
    src_e = jnp.concatenate([
        p_edge_index[0].astype(i32), v_edge_index[0].astype(i32) + V_BASE])
    dst_e = jnp.concatenate([
        p_edge_index[1].astype(i32), v_edge_index[1].astype(i32) + V_BASE])
    dstT_deg = jnp.concatenate([
        dst_e, jnp.full((E_PAD - E_ALL,), PAD_NODE, i32)]).reshape(NW, NT, CH)
    pad_c = jnp.full((CE_PAD - E_ALL,), PAD_NODE, i32)
    src_all = jnp.concatenate([src_e, pad_c])
    dst_all = jnp.concatenate([dst_e, pad_c])

    linW = jnp.stack([p_lin_W, v_lin_W])
    linb = jnp.stack([p_lin_b, v_lin_b])[:, None, :]
    g1W = jnp.stack([p_g1_W, v_g1_W])
    b1 = jnp.stack([p_g1_b, v_g1_b])[:, None, :]
    g2W = jnp.stack([p_g2_W, v_g2_W])
    b2 = jnp.stack([p_g2_b, v_g2_b])[:, None, :]

    deg_bins = _sc_degrees(dstT_deg)

    h0, y1, dinv = _tc_a(x_all, linW, linb, g1W, deg_bins)
    S1 = _sc_edge_aggregate(y1, src_all, dst_all)
    y2 = _tc_b(S1, y1, dinv, g2W, b1)
    S2 = _sc_edge_aggregate(y2, src_all, dst_all)
    h2 = _tc_c(S2, y2, dinv, b2)

    h2f = jnp.pad(h2, ((0, FIN_ROWS - NODES), (0, 0)))
    h0f = jnp.pad(h0, ((0, FIN_ROWS - NODES), (0, 0)))
    neg = jnp.full((V_BASE - P_N,), -1.0, f32)
    pbf = jnp.concatenate([
        p_batch.astype(f32), neg,
        jnp.full((FIN_ROWS - V_BASE,), -1.0, f32)])[None]
    vbf = jnp.concatenate([
        jnp.full((V_BASE,), -1.0, f32), v_batch.astype(f32),
        jnp.full((FIN_ROWS - NODES,), -1.0, f32)])[None]

    fusion, compat, att_mask = _finalize(h2f, h0f, pbf, vbf, att_Wq, att_Wk)
    return fusion[:, 0, :], compat, att_mask


# speedup vs baseline: 16.8549x; 1.0888x over previous
import functools

import jax
import jax.numpy as jnp
import numpy as np
from jax import lax
from jax.experimental import pallas as pl
from jax.experimental.pallas import tpu as pltpu
from jax.experimental.pallas import tpu_sc as plsc

B = 8
EMB = 128
H = 4
P_N, P_E = 10000, 320000
V_N, V_E = 2048, 8192
P_MAX, V_MAX = 2048, 384

PAD_NODE = P_N
V_BASE = 10240
NODES = V_BASE + V_N
FIN_ROWS = V_BASE + V_N + V_MAX

NC, NS, L = 2, 16, 16
NW = NC * NS
E_ALL = P_E + V_E
ROWS_PER_TILE = NODES // NS

CH = 128
T = 2592
E_PAD = T * CH
NT = T // NW

CCH = 112
CNT = 92
CE_PAD = NW * CNT * CCH


def _sc_mesh():
    return plsc.VectorSubcoreMesh(core_axis_name="c", subcore_axis_name="s",
                                  num_cores=NC, num_subcores=NS)


def _zero_fill_128w(zbuf):
    z16 = jnp.zeros((L,), jnp.float32)

    def body(i, _):
        for g in range(8):
            zbuf[i, pl.ds(g * L, L)] = z16
        return 0
    lax.fori_loop(0, 128, body, 0)


def _deg_body(dstT, deg_out, bins, buf, didx):
    c = lax.axis_index("c")
    s = lax.axis_index("s")
    wid = s * NC + c

    _zero_fill_128w(buf)
    r0 = s * ROWS_PER_TILE
    for j in range(ROWS_PER_TILE // 128):
        pltpu.sync_copy(buf, bins.at[pl.ds(r0 + j * 128, 128)])

    one16 = jnp.full((L,), 1.0, jnp.float32)

    def fill(i, _):
        for g in range(EMB // L):
            buf[i, pl.ds(g * L, L)] = one16
        return 0
    lax.fori_loop(0, 128, fill, 0)
    plsc.subcore_barrier()

    pltpu.sync_copy(dstT.at[wid], didx)

    def scat(j, _):
        pltpu.sync_copy(buf, bins.at[didx.at[j]], add=True)
        return 0
    lax.fori_loop(0, NT, scat, 0)
    plsc.subcore_barrier()

    pltpu.sync_copy(bins.at[pl.ds(r0, ROWS_PER_TILE)],
                    deg_out.at[c, pl.ds(r0, ROWS_PER_TILE)])


def _sc_degrees(dstT):
    kern = pl.kernel(
        _deg_body,
        out_type=jax.ShapeDtypeStruct((NC, NODES, EMB), jnp.float32),
        mesh=_sc_mesh(),
        scratch_types=[
            pltpu.VMEM_SHARED((NODES, EMB), jnp.float32),
            pltpu.VMEM((128, EMB), jnp.float32),
            pltpu.VMEM((NT, CH), jnp.int32),
        ],
    )
    return kern(dstT)


def _conv_body(y, src1d, dst1d, S_out, acc,
               rows0, rows1, sidx0, sidx1, didx0, didx1, sem0, sem1, isem):
    c = lax.axis_index("c")
    s = lax.axis_index("s")
    wid = s * NC + c

    def zfill(i, _):
        for g in range(EMB // L):
            rows0[i, pl.ds(g * L, L)] = jnp.zeros((L,), jnp.float32)
        return 0
    lax.fori_loop(0, CCH, zfill, 0)
    r0 = s * ROWS_PER_TILE
    for off in range(0, ROWS_PER_TILE, CCH):
        size = min(CCH, ROWS_PER_TILE - off)
        pltpu.sync_copy(rows0.at[pl.ds(0, size)],
                        acc.at[pl.ds(r0 + off, size)])
    plsc.subcore_barrier()

    base = wid * CNT * CCH

    def fetch_sidx(j, sidx):
        pltpu.async_copy(src1d.at[pl.ds(base + j * CCH, CCH)], sidx, isem)

    def fetch_didx(j, didx):
        pltpu.async_copy(dst1d.at[pl.ds(base + j * CCH, CCH)], didx, isem)

    def wait_one_idx(sidx):
        pltpu.make_async_copy(src1d.at[pl.ds(base, CCH)], sidx, isem).wait()

    fetch_sidx(0, sidx0)
    fetch_didx(0, didx0)
    wait_one_idx(sidx0)
    wait_one_idx(didx0)
    pltpu.async_copy(y.at[sidx0], rows0, sem0)
    fetch_sidx(1, sidx1)
    fetch_didx(1, didx1)

    def work(i, sidx_c, didx_c, rows_c, sem_c, sidx_n, didx_n, rows_n, sem_n):
        @pl.when(i + 1 < CNT)
        def _():
            wait_one_idx(sidx_n)
            wait_one_idx(didx_n)
            pltpu.async_copy(y.at[sidx_n], rows_n, sem_n)
        pltpu.make_async_copy(y.at[sidx_c], rows_c, sem_c).wait()
        @pl.when(i + 2 < CNT)
        def _():
            fetch_sidx(i + 2, sidx_c)
        pltpu.sync_copy(rows_c, acc.at[didx_c], add=True)
        @pl.when(i + 2 < CNT)
        def _():
            fetch_didx(i + 2, didx_c)

    def step(i, _):
        @pl.when(i % 2 == 0)
        def _():
            work(i, sidx0, didx0, rows0, sem0, sidx1, didx1, rows1, sem1)
        @pl.when(i % 2 == 1)
        def _():
            work(i, sidx1, didx1, rows1, sem1, sidx0, didx0, rows0, sem0)
        return 0
    lax.fori_loop(0, CNT, step, 0)
    plsc.subcore_barrier()

    pltpu.sync_copy(acc.at[pl.ds(r0, ROWS_PER_TILE)],
                    S_out.at[c, pl.ds(r0, ROWS_PER_TILE)])


def _sc_edge_aggregate(y, src1d, dst1d):
    kern = pl.kernel(
        _conv_body,
        out_type=jax.ShapeDtypeStruct((NC, NODES, EMB), jnp.float32),
        mesh=_sc_mesh(),
        scratch_types=[
            pltpu.VMEM_SHARED((NODES, EMB), jnp.float32),
            pltpu.VMEM((CCH, EMB), jnp.float32),
            pltpu.VMEM((CCH, EMB), jnp.float32),
            pltpu.VMEM((CCH,), jnp.int32),
            pltpu.VMEM((CCH,), jnp.int32),
            pltpu.VMEM((CCH,), jnp.int32),
            pltpu.VMEM((CCH,), jnp.int32),
            pltpu.SemaphoreType.DMA,
            pltpu.SemaphoreType.DMA,
            pltpu.SemaphoreType.DMA,
        ],
    )
    return kern(y, src1d, dst1d)


NODE_BLK = 1024
N_BLKS = NODES // NODE_BLK
P_BLKS = V_BASE // NODE_BLK


def _wsel(k):
    return jnp.where(k < P_BLKS, 0, 1)


def _tc_a_body(x_ref, linW_ref, linb_ref, g1W_ref, deg_ref,
               h0_ref, y1_ref, dinv_ref):
    x = x_ref[...]
    h0 = jax.lax.dot_general(x, linW_ref[0], (((1,), (0,)), ((), ())),
                             preferred_element_type=jnp.float32) + linb_ref[0]
    deg = (jnp.sum(deg_ref[0] + deg_ref[1], axis=-1, keepdims=True)
           * (1.0 / EMB) + 1.0)
    dinv = lax.rsqrt(jnp.maximum(deg, 1.0))
    xw = jax.lax.dot_general(h0, g1W_ref[0], (((1,), (0,)), ((), ())),
                             preferred_element_type=jnp.float32)
    h0_ref[...] = h0
    y1_ref[...] = xw * dinv
    dinv_ref[...] = dinv


def _tc_a(x_all, linW, linb, g1W, deg):
    return pl.pallas_call(
        _tc_a_body,
        grid=(N_BLKS,),
        in_specs=[
            pl.BlockSpec((NODE_BLK, EMB), lambda k: (k, 0)),
            pl.BlockSpec((1, EMB, EMB), lambda k: (_wsel(k), 0, 0)),
            pl.BlockSpec((1, 1, EMB), lambda k: (_wsel(k), 0, 0)),
            pl.BlockSpec((1, EMB, EMB), lambda k: (_wsel(k), 0, 0)),
            pl.BlockSpec((2, NODE_BLK, EMB), lambda k: (0, k, 0)),
        ],
        out_specs=[
            pl.BlockSpec((NODE_BLK, EMB), lambda k: (k, 0)),
            pl.BlockSpec((NODE_BLK, EMB), lambda k: (k, 0)),
            pl.BlockSpec((NODE_BLK, 1), lambda k: (k, 0)),
        ],
        out_shape=[
            jax.ShapeDtypeStruct((NODES, EMB), jnp.float32),
            jax.ShapeDtypeStruct((NODES, EMB), jnp.float32),
            jax.ShapeDtypeStruct((NODES, 1), jnp.float32),
        ],
    )(x_all, linW, linb, g1W, deg)


def _tc_b_body(S_ref, y1_ref, dinv_ref, g2W_ref, b1_ref, y2_ref):
    dinv = dinv_ref[...]
    h1 = jax.nn.relu(dinv * (S_ref[0] + S_ref[1] + y1_ref[...]) + b1_ref[0])
    xw = jax.lax.dot_general(h1, g2W_ref[0], (((1,), (0,)), ((), ())),
                             preferred_element_type=jnp.float32)
    y2_ref[...] = xw * dinv


def _tc_b(S1, y1, dinv, g2W, b1):
    return pl.pallas_call(
        _tc_b_body,
        grid=(N_BLKS,),
        in_specs=[
            pl.BlockSpec((2, NODE_BLK, EMB), lambda k: (0, k, 0)),
            pl.BlockSpec((NODE_BLK, EMB), lambda k: (k, 0)),
            pl.BlockSpec((NODE_BLK, 1), lambda k: (k, 0)),
            pl.BlockSpec((1, EMB, EMB), lambda k: (_wsel(k), 0, 0)),
            pl.BlockSpec((1, 1, EMB), lambda k: (_wsel(k), 0, 0)),
        ],
        out_specs=pl.BlockSpec((NODE_BLK, EMB), lambda k: (k, 0)),
        out_shape=jax.ShapeDtypeStruct((NODES, EMB), jnp.float32),
    )(S1, y1, dinv, g2W, b1)


def _tc_c_body(S_ref, y2_ref, dinv_ref, b2_ref, h2_ref):
    h2_ref[...] = (dinv_ref[...] * (S_ref[0] + S_ref[1] + y2_ref[...])
                   + b2_ref[0])


def _tc_c(S2, y2, dinv, b2):
    return pl.pallas_call(
        _tc_c_body,
        grid=(N_BLKS,),
        in_specs=[
            pl.BlockSpec((2, NODE_BLK, EMB), lambda k: (0, k, 0)),
            pl.BlockSpec((NODE_BLK, EMB), lambda k: (k, 0)),
            pl.BlockSpec((NODE_BLK, 1), lambda k: (k, 0)),
            pl.BlockSpec((1, 1, EMB), lambda k: (_wsel(k), 0, 0)),
        ],
        out_specs=pl.BlockSpec((NODE_BLK, EMB), lambda k: (k, 0)),
        out_shape=jax.ShapeDtypeStruct((NODES, EMB), jnp.float32),
    )(S2, y2, dinv, b2)


def _finalize_body(h2_ref, h0_ref, pb_ref, vb_ref, wq_ref, wk_ref,
                   fusion_ref, compat_ref, mask_ref):
    b = pl.program_id(0)
    bf = jnp.float32(b)

    pb = pb_ref[...]
    vb = vb_ref[...]
    pm = (pb == bf).astype(jnp.float32)
    vm = (vb == bf).astype(jnp.float32)
    p_cnt = jnp.sum(pm)
    v_cnt = jnp.sum(vm)
    p_start = jnp.sum((pb >= 0.0) & (pb < bf)).astype(jnp.int32)
    v_start = jnp.sum((vb >= 0.0) & (vb < bf)).astype(jnp.int32) + V_BASE

    h2 = h2_ref[...]
    p_g = jax.lax.dot_general(pm, h2, (((1,), (0,)), ((), ())),
                              preferred_element_type=jnp.float32
                              ) / jnp.maximum(p_cnt, 1.0)
    v_g = jax.lax.dot_general(vm, h2, (((1,), (0,)), ((), ())),
                              preferred_element_type=jnp.float32
                              ) / jnp.maximum(v_cnt, 1.0)
    fusion_ref[...] = ((p_g + v_g) / 2.0)[None]

    p_rows = h2_ref[pl.ds(p_start, P_MAX), :] + h0_ref[pl.ds(p_start, P_MAX), :]
    v_rows = h2_ref[pl.ds(v_start, V_MAX), :] + h0_ref[pl.ds(v_start, V_MAX), :]
    p_cnt_i = jnp.minimum(p_cnt, float(P_MAX)).astype(jnp.int32)
    v_cnt_i = jnp.minimum(v_cnt, float(V_MAX)).astype(jnp.int32)
    p_valid = (jax.lax.broadcasted_iota(jnp.int32, (P_MAX, 1), 0)
               < p_cnt_i).astype(jnp.float32)
    v_valid = (jax.lax.broadcasted_iota(jnp.int32, (V_MAX, 1), 0)
               < v_cnt_i).astype(jnp.float32)
    P_blk = p_rows * p_valid + p_g
    V_blk = v_rows * v_valid + v_g

    dh = EMB // H
    scale = 1.0 / (H * np.sqrt(dh))
    A = jax.lax.dot_general(wq_ref[...], wk_ref[...], (((1,), (1,)), ((), ())),
                            preferred_element_type=jnp.float32) * scale
    PA = jax.lax.dot_general(P_blk, A, (((1,), (0,)), ((), ())),
                             preferred_element_type=jnp.float32)
    compat = jax.lax.dot_general(PA, V_blk, (((1,), (1,)), ((), ())),
                                 preferred_element_type=jnp.float32)
    compat_ref[...] = compat[None]

    v_iota_row = jax.lax.broadcasted_iota(jnp.int32, (P_MAX, V_MAX), 1)
    mask_ref[...] = (v_iota_row < v_cnt_i)[None]


def _finalize(h2f, h0f, pbf, vbf, att_Wq, att_Wk):
    full = lambda shape: pl.BlockSpec(shape, lambda b: (0,) * len(shape))
    return pl.pallas_call(
        _finalize_body,
        grid=(B,),
        in_specs=[
            full((FIN_ROWS, EMB)), full((FIN_ROWS, EMB)),
            full((1, FIN_ROWS)), full((1, FIN_ROWS)),
            full((EMB, EMB)), full((EMB, EMB)),
        ],
        out_specs=[
            pl.BlockSpec((1, 1, EMB), lambda b: (b, 0, 0)),
            pl.BlockSpec((1, P_MAX, V_MAX), lambda b: (b, 0, 0)),
            pl.BlockSpec((1, P_MAX, V_MAX), lambda b: (b, 0, 0)),
        ],
        out_shape=[
            jax.ShapeDtypeStruct((B, 1, EMB), jnp.float32),
            jax.ShapeDtypeStruct((B, P_MAX, V_MAX), jnp.float32),
            jax.ShapeDtypeStruct((B, P_MAX, V_MAX), jnp.bool_),
        ],
    )(h2f, h0f, pbf, vbf, att_Wq, att_Wk)


def kernel(p_x, v_x, p_lin_W, p_lin_b, p_g1_W, p_g1_b, p_g2_W, p_g2_b,
           v_lin_W, v_lin_b, v_g1_W, v_g1_b, v_g2_W, v_g2_b,
           att_Wq, att_Wk, p_edge_index, p_batch, v_edge_index, v_batch):
    f32 = jnp.float32
    i32 = jnp.int32

    x_all = jnp.concatenate([
        jnp.pad(p_x, ((0, V_BASE - P_N), (0, 0))), v_x], axis=0)
    src_e = jnp.concatenate([
        p_edge_index[0].astype(i32), v_edge_index[0].astype(i32) + V_BASE])
    dst_e = jnp.concatenate([
        p_edge_index[1].astype(i32), v_edge_index[1].astype(i32) + V_BASE])
    dstT_deg = jnp.concatenate([
        dst_e, jnp.full((E_PAD - E_ALL,), PAD_NODE, i32)]).reshape(NW, NT, CH)
    pad_c = jnp.full((CE_PAD - E_ALL,), PAD_NODE, i32)
    src_all = jnp.concatenate([src_e, pad_c])
    dst_all = jnp.concatenate([dst_e, pad_c])

    linW = jnp.stack([p_lin_W, v_lin_W])
    linb = jnp.stack([p_lin_b, v_lin_b])[:, None, :]
    g1W = jnp.stack([p_g1_W, v_g1_W])
    b1 = jnp.stack([p_g1_b, v_g1_b])[:, None, :]
    g2W = jnp.stack([p_g2_W, v_g2_W])
    b2 = jnp.stack([p_g2_b, v_g2_b])[:, None, :]

    deg_bins = _sc_degrees(dstT_deg)

    h0, y1, dinv = _tc_a(x_all, linW, linb, g1W, deg_bins)
    S1 = _sc_edge_aggregate(y1, src_all, dst_all)
    y2 = _tc_b(S1, y1, dinv, g2W, b1)
    S2 = _sc_edge_aggregate(y2, src_all, dst_all)
    h2 = _tc_c(S2, y2, dinv, b2)

    h2f = jnp.pad(h2, ((0, FIN_ROWS - NODES), (0, 0)))
    h0f = jnp.pad(h0, ((0, FIN_ROWS - NODES), (0, 0)))
    neg = jnp.full((V_BASE - P_N,), -1.0, f32)
    pbf = jnp.concatenate([
        p_batch.astype(f32), neg,
        jnp.full((FIN_ROWS - V_BASE,), -1.0, f32)])[None]
    vbf = jnp.concatenate([
        jnp.full((V_BASE,), -1.0, f32), v_batch.astype(f32),
        jnp.full((FIN_ROWS - NODES,), -1.0, f32)])[None]

    fusion, compat, att_mask = _finalize(h2f, h0f, pbf, vbf, att_Wq, att_Wk)
    return fusion[:, 0, :], compat, att_mask


# speedup vs baseline: 17.1651x; 1.0184x over previous
import functools

import jax
import jax.numpy as jnp
import numpy as np
from jax import lax
from jax.experimental import pallas as pl
from jax.experimental.pallas import tpu as pltpu
from jax.experimental.pallas import tpu_sc as plsc

B = 8
EMB = 128
H = 4
P_N, P_E = 10000, 320000
V_N, V_E = 2048, 8192
P_MAX, V_MAX = 2048, 384

PAD_NODE = P_N
V_BASE = 10240
NODES = V_BASE + V_N
FIN_ROWS = V_BASE + V_N + V_MAX

NC, NS, L = 2, 16, 16
NW = NC * NS
E_ALL = P_E + V_E
ROWS_PER_TILE = NODES // NS

CH = 128
T = 2592
E_PAD = T * CH
NT = T // NW

CCH = 112
CNT = 92
CE_PAD = NW * CNT * CCH


def _sc_mesh():
    return plsc.VectorSubcoreMesh(core_axis_name="c", subcore_axis_name="s",
                                  num_cores=NC, num_subcores=NS)


def _zero_fill_128w(zbuf):
    z16 = jnp.zeros((L,), jnp.float32)

    def body(i, _):
        for g in range(8):
            zbuf[i, pl.ds(g * L, L)] = z16
        return 0
    lax.fori_loop(0, 128, body, 0)


def _deg_body(dstT, deg_out, bins, buf, didx, sem):
    c = lax.axis_index("c")
    s = lax.axis_index("s")
    wid = s * NC + c

    _zero_fill_128w(buf)
    r0 = s * ROWS_PER_TILE
    for j in range(ROWS_PER_TILE // 128):
        pltpu.sync_copy(buf, bins.at[pl.ds(r0 + j * 128, 128)])

    one16 = jnp.full((L,), 1.0, jnp.float32)

    def fill(i, _):
        for g in range(EMB // L):
            buf[i, pl.ds(g * L, L)] = one16
        return 0
    lax.fori_loop(0, 128, fill, 0)
    plsc.subcore_barrier()

    pltpu.sync_copy(dstT.at[wid], didx)
    W = 8

    def scat(j, _):
        pltpu.async_copy(buf, bins.at[didx.at[j]], sem, add=True)
        @pl.when(j >= W - 1)
        def _():
            pltpu.make_async_copy(buf, bins.at[didx.at[0]], sem).wait()
        return 0
    lax.fori_loop(0, NT, scat, 0)
    for _ in range(W - 1):
        pltpu.make_async_copy(buf, bins.at[didx.at[0]], sem).wait()
    plsc.subcore_barrier()

    pltpu.sync_copy(bins.at[pl.ds(r0, ROWS_PER_TILE)],
                    deg_out.at[c, pl.ds(r0, ROWS_PER_TILE)])


def _sc_degrees(dstT):
    kern = pl.kernel(
        _deg_body,
        out_type=jax.ShapeDtypeStruct((NC, NODES, EMB), jnp.float32),
        mesh=_sc_mesh(),
        scratch_types=[
            pltpu.VMEM_SHARED((NODES, EMB), jnp.float32),
            pltpu.VMEM((128, EMB), jnp.float32),
            pltpu.VMEM((NT, CH), jnp.int32),
            pltpu.SemaphoreType.DMA,
        ],
    )
    return kern(dstT)


def _conv_body(y, src1d, dst1d, S_out, acc,
               rows0, rows1, sidx0, sidx1, didx0, didx1, sem0, sem1, isem):
    c = lax.axis_index("c")
    s = lax.axis_index("s")
    wid = s * NC + c

    def zfill(i, _):
        for g in range(EMB // L):
            rows0[i, pl.ds(g * L, L)] = jnp.zeros((L,), jnp.float32)
        return 0
    lax.fori_loop(0, CCH, zfill, 0)
    r0 = s * ROWS_PER_TILE
    for off in range(0, ROWS_PER_TILE, CCH):
        size = min(CCH, ROWS_PER_TILE - off)
        pltpu.sync_copy(rows0.at[pl.ds(0, size)],
                        acc.at[pl.ds(r0 + off, size)])
    plsc.subcore_barrier()

    base = wid * CNT * CCH

    def fetch_sidx(j, sidx):
        pltpu.async_copy(src1d.at[pl.ds(base + j * CCH, CCH)], sidx, isem)

    def fetch_didx(j, didx):
        pltpu.async_copy(dst1d.at[pl.ds(base + j * CCH, CCH)], didx, isem)

    def wait_one_idx(sidx):
        pltpu.make_async_copy(src1d.at[pl.ds(base, CCH)], sidx, isem).wait()

    fetch_sidx(0, sidx0)
    fetch_didx(0, didx0)
    wait_one_idx(sidx0)
    wait_one_idx(didx0)
    pltpu.async_copy(y.at[sidx0], rows0, sem0)
    fetch_sidx(1, sidx1)
    fetch_didx(1, didx1)

    def work(i, sidx_c, didx_c, rows_c, sem_c, sidx_n, didx_n, rows_n, sem_n):
        @pl.when(i + 1 < CNT)
        def _():
            wait_one_idx(sidx_n)
            wait_one_idx(didx_n)
            pltpu.async_copy(y.at[sidx_n], rows_n, sem_n)
        pltpu.make_async_copy(y.at[sidx_c], rows_c, sem_c).wait()
        @pl.when(i + 2 < CNT)
        def _():
            fetch_sidx(i + 2, sidx_c)
        pltpu.sync_copy(rows_c, acc.at[didx_c], add=True)
        @pl.when(i + 2 < CNT)
        def _():
            fetch_didx(i + 2, didx_c)

    def step(i, _):
        @pl.when(i % 2 == 0)
        def _():
            work(i, sidx0, didx0, rows0, sem0, sidx1, didx1, rows1, sem1)
        @pl.when(i % 2 == 1)
        def _():
            work(i, sidx1, didx1, rows1, sem1, sidx0, didx0, rows0, sem0)
        return 0
    lax.fori_loop(0, CNT, step, 0)
    plsc.subcore_barrier()

    pltpu.sync_copy(acc.at[pl.ds(r0, ROWS_PER_TILE)],
                    S_out.at[c, pl.ds(r0, ROWS_PER_TILE)])


def _sc_edge_aggregate(y, src1d, dst1d):
    kern = pl.kernel(
        _conv_body,
        out_type=jax.ShapeDtypeStruct((NC, NODES, EMB), jnp.float32),
        mesh=_sc_mesh(),
        scratch_types=[
            pltpu.VMEM_SHARED((NODES, EMB), jnp.float32),
            pltpu.VMEM((CCH, EMB), jnp.float32),
            pltpu.VMEM((CCH, EMB), jnp.float32),
            pltpu.VMEM((CCH,), jnp.int32),
            pltpu.VMEM((CCH,), jnp.int32),
            pltpu.VMEM((CCH,), jnp.int32),
            pltpu.VMEM((CCH,), jnp.int32),
            pltpu.SemaphoreType.DMA,
            pltpu.SemaphoreType.DMA,
            pltpu.SemaphoreType.DMA,
        ],
    )
    return kern(y, src1d, dst1d)


NODE_BLK = 1024
N_BLKS = NODES // NODE_BLK
P_BLKS = V_BASE // NODE_BLK


def _wsel(k):
    return jnp.where(k < P_BLKS, 0, 1)


def _tc_a_body(x_ref, linW_ref, linb_ref, g1W_ref, deg_ref,
               h0_ref, y1_ref, dinv_ref):
    x = x_ref[...]
    h0 = jax.lax.dot_general(x, linW_ref[0], (((1,), (0,)), ((), ())),
                             preferred_element_type=jnp.float32) + linb_ref[0]
    deg = (jnp.sum(deg_ref[0] + deg_ref[1], axis=-1, keepdims=True)
           * (1.0 / EMB) + 1.0)
    dinv = lax.rsqrt(jnp.maximum(deg, 1.0))
    xw = jax.lax.dot_general(h0, g1W_ref[0], (((1,), (0,)), ((), ())),
                             preferred_element_type=jnp.float32)
    h0_ref[...] = h0
    y1_ref[...] = xw * dinv
    dinv_ref[...] = dinv


def _tc_a(x_all, linW, linb, g1W, deg):
    return pl.pallas_call(
        _tc_a_body,
        grid=(N_BLKS,),
        in_specs=[
            pl.BlockSpec((NODE_BLK, EMB), lambda k: (k, 0)),
            pl.BlockSpec((1, EMB, EMB), lambda k: (_wsel(k), 0, 0)),
            pl.BlockSpec((1, 1, EMB), lambda k: (_wsel(k), 0, 0)),
            pl.BlockSpec((1, EMB, EMB), lambda k: (_wsel(k), 0, 0)),
            pl.BlockSpec((2, NODE_BLK, EMB), lambda k: (0, k, 0)),
        ],
        out_specs=[
            pl.BlockSpec((NODE_BLK, EMB), lambda k: (k, 0)),
            pl.BlockSpec((NODE_BLK, EMB), lambda k: (k, 0)),
            pl.BlockSpec((NODE_BLK, 1), lambda k: (k, 0)),
        ],
        out_shape=[
            jax.ShapeDtypeStruct((NODES, EMB), jnp.float32),
            jax.ShapeDtypeStruct((NODES, EMB), jnp.float32),
            jax.ShapeDtypeStruct((NODES, 1), jnp.float32),
        ],
    )(x_all, linW, linb, g1W, deg)


def _tc_b_body(S_ref, y1_ref, dinv_ref, g2W_ref, b1_ref, y2_ref):
    dinv = dinv_ref[...]
    h1 = jax.nn.relu(dinv * (S_ref[0] + S_ref[1] + y1_ref[...]) + b1_ref[0])
    xw = jax.lax.dot_general(h1, g2W_ref[0], (((1,), (0,)), ((), ())),
                             preferred_element_type=jnp.float32)
    y2_ref[...] = xw * dinv


def _tc_b(S1, y1, dinv, g2W, b1):
    return pl.pallas_call(
        _tc_b_body,
        grid=(N_BLKS,),
        in_specs=[
            pl.BlockSpec((2, NODE_BLK, EMB), lambda k: (0, k, 0)),
            pl.BlockSpec((NODE_BLK, EMB), lambda k: (k, 0)),
            pl.BlockSpec((NODE_BLK, 1), lambda k: (k, 0)),
            pl.BlockSpec((1, EMB, EMB), lambda k: (_wsel(k), 0, 0)),
            pl.BlockSpec((1, 1, EMB), lambda k: (_wsel(k), 0, 0)),
        ],
        out_specs=pl.BlockSpec((NODE_BLK, EMB), lambda k: (k, 0)),
        out_shape=jax.ShapeDtypeStruct((NODES, EMB), jnp.float32),
    )(S1, y1, dinv, g2W, b1)


def _finalize_body(S_ref, y2_ref, dinv_ref, b2_ref, h0_ref,
                   pb_ref, vb_ref, wq_ref, wk_ref,
                   fusion_ref, compat_ref, mask_ref, hsum_ref):
    b = pl.program_id(0)
    bf = jnp.float32(b)

    @pl.when(b == 0)
    def _():
        row = jax.lax.broadcasted_iota(jnp.int32, (NODES, 1), 0)
        bias = jnp.where(row < V_BASE, b2_ref[0], b2_ref[1])
        h2 = dinv_ref[...] * (S_ref[0] + S_ref[1] + y2_ref[...]) + bias
        hsum_ref[pl.ds(0, NODES), :] = h2 + h0_ref[...]
        hsum_ref[pl.ds(NODES, FIN_ROWS - NODES), :] = jnp.zeros(
            (FIN_ROWS - NODES, EMB), jnp.float32)

    pb = pb_ref[...]
    vb = vb_ref[...]
    pm = (pb == bf).astype(jnp.float32)
    vm = (vb == bf).astype(jnp.float32)
    p_cnt = jnp.sum(pm)
    v_cnt = jnp.sum(vm)
    p_start = jnp.sum((pb >= 0.0) & (pb < bf)).astype(jnp.int32)
    v_start = jnp.sum((vb >= 0.0) & (vb < bf)).astype(jnp.int32) + V_BASE

    hsum_all = hsum_ref[...]
    h0 = h0_ref[...]
    p_g = (jax.lax.dot_general(pm, hsum_all, (((1,), (0,)), ((), ())),
                               preferred_element_type=jnp.float32)
           - jax.lax.dot_general(pm[:, :NODES], h0, (((1,), (0,)), ((), ())),
                                 preferred_element_type=jnp.float32)
           ) / jnp.maximum(p_cnt, 1.0)
    v_g = (jax.lax.dot_general(vm, hsum_all, (((1,), (0,)), ((), ())),
                               preferred_element_type=jnp.float32)
           - jax.lax.dot_general(vm[:, :NODES], h0, (((1,), (0,)), ((), ())),
                                 preferred_element_type=jnp.float32)
           ) / jnp.maximum(v_cnt, 1.0)
    fusion_ref[...] = ((p_g + v_g) / 2.0)[None]

    p_rows = hsum_ref[pl.ds(p_start, P_MAX), :]
    v_rows = hsum_ref[pl.ds(v_start, V_MAX), :]
    p_cnt_i = jnp.minimum(p_cnt, float(P_MAX)).astype(jnp.int32)
    v_cnt_i = jnp.minimum(v_cnt, float(V_MAX)).astype(jnp.int32)
    p_valid = (jax.lax.broadcasted_iota(jnp.int32, (P_MAX, 1), 0)
               < p_cnt_i).astype(jnp.float32)
    v_valid = (jax.lax.broadcasted_iota(jnp.int32, (V_MAX, 1), 0)
               < v_cnt_i).astype(jnp.float32)
    P_blk = p_rows * p_valid + p_g
    V_blk = v_rows * v_valid + v_g

    dh = EMB // H
    scale = 1.0 / (H * np.sqrt(dh))
    A = jax.lax.dot_general(wq_ref[...], wk_ref[...], (((1,), (1,)), ((), ())),
                            preferred_element_type=jnp.float32) * scale
    PA = jax.lax.dot_general(P_blk, A, (((1,), (0,)), ((), ())),
                             preferred_element_type=jnp.float32)
    compat = jax.lax.dot_general(PA, V_blk, (((1,), (1,)), ((), ())),
                                 preferred_element_type=jnp.float32)
    compat_ref[...] = compat[None]

    v_iota_row = jax.lax.broadcasted_iota(jnp.int32, (P_MAX, V_MAX), 1)
    mask_ref[...] = (v_iota_row < v_cnt_i)[None]


def _finalize(S2, y2, dinv, b2, h0, pbf, vbf, att_Wq, att_Wk):
    full = lambda shape: pl.BlockSpec(shape, lambda b: (0,) * len(shape))
    return pl.pallas_call(
        _finalize_body,
        grid=(B,),
        in_specs=[
            full((2, NODES, EMB)), full((NODES, EMB)), full((NODES, 1)),
            full((2, 1, EMB)), full((NODES, EMB)),
            full((1, FIN_ROWS)), full((1, FIN_ROWS)),
            full((EMB, EMB)), full((EMB, EMB)),
        ],
        out_specs=[
            pl.BlockSpec((1, 1, EMB), lambda b: (b, 0, 0)),
            pl.BlockSpec((1, P_MAX, V_MAX), lambda b: (b, 0, 0)),
            pl.BlockSpec((1, P_MAX, V_MAX), lambda b: (b, 0, 0)),
        ],
        out_shape=[
            jax.ShapeDtypeStruct((B, 1, EMB), jnp.float32),
            jax.ShapeDtypeStruct((B, P_MAX, V_MAX), jnp.float32),
            jax.ShapeDtypeStruct((B, P_MAX, V_MAX), jnp.bool_),
        ],
        scratch_shapes=[pltpu.VMEM((FIN_ROWS, EMB), jnp.float32)],
    )(S2, y2, dinv, b2, h0, pbf, vbf, att_Wq, att_Wk)


def kernel(p_x, v_x, p_lin_W, p_lin_b, p_g1_W, p_g1_b, p_g2_W, p_g2_b,
           v_lin_W, v_lin_b, v_g1_W, v_g1_b, v_g2_W, v_g2_b,
           att_Wq, att_Wk, p_edge_index, p_batch, v_edge_index, v_batch):
    f32 = jnp.float32
    i32 = jnp.int32

    x_all = jnp.concatenate([
        jnp.pad(p_x, ((0, V_BASE - P_N), (0, 0))), v_x], axis=0)
    src_e = jnp.concatenate([
        p_edge_index[0].astype(i32), v_edge_index[0].astype(i32) + V_BASE])
    dst_e = jnp.concatenate([
        p_edge_index[1].astype(i32), v_edge_index[1].astype(i32) + V_BASE])
    dstT_deg = jnp.concatenate([
        dst_e, jnp.full((E_PAD - E_ALL,), PAD_NODE, i32)]).reshape(NW, NT, CH)
    pad_c = jnp.full((CE_PAD - E_ALL,), PAD_NODE, i32)
    src_all = jnp.concatenate([src_e, pad_c])
    dst_all = jnp.concatenate([dst_e, pad_c])

    linW = jnp.stack([p_lin_W, v_lin_W])
    linb = jnp.stack([p_lin_b, v_lin_b])[:, None, :]
    g1W = jnp.stack([p_g1_W, v_g1_W])
    b1 = jnp.stack([p_g1_b, v_g1_b])[:, None, :]
    g2W = jnp.stack([p_g2_W, v_g2_W])
    b2 = jnp.stack([p_g2_b, v_g2_b])[:, None, :]

    deg_bins = _sc_degrees(dstT_deg)

    h0, y1, dinv = _tc_a(x_all, linW, linb, g1W, deg_bins)
    S1 = _sc_edge_aggregate(y1, src_all, dst_all)
    y2 = _tc_b(S1, y1, dinv, g2W, b1)
    S2 = _sc_edge_aggregate(y2, src_all, dst_all)

    neg = jnp.full((V_BASE - P_N,), -1.0, f32)
    pbf = jnp.concatenate([
        p_batch.astype(f32), neg,
        jnp.full((FIN_ROWS - V_BASE,), -1.0, f32)])[None]
    vbf = jnp.concatenate([
        jnp.full((V_BASE,), -1.0, f32), v_batch.astype(f32),
        jnp.full((FIN_ROWS - NODES,), -1.0, f32)])[None]

    fusion, compat, att_mask = _finalize(S2, y2, dinv, b2, h0,
                                         pbf, vbf, att_Wq, att_Wk)
    return fusion[:, 0, :], compat, att_mask
